# baseline (speedup 1.0000x reference)
import functools

import jax
import jax.numpy as jnp
from jax import lax
from jax.experimental import pallas as pl
from jax.experimental.pallas import tpu as pltpu
from jax.experimental.pallas import tpu_sc as plsc

_N = 10000
_NPAD = 10240
_E = 320000
_EPAD = 327680
_EPT = _EPAD // 32
_NCHUNK = _EPT // 128
_NGRP = _NCHUNK // 4
_RPT = _NPAD // 16

_mesh = plsc.VectorSubcoreMesh(core_axis_name="c", subcore_axis_name="s")
_sc_params = pltpu.CompilerParams(use_tc_tiling_on_sc=False)


def _zero_accum(zbuf, accum, si, width):
  zero16 = jnp.zeros((16,), jnp.float32)
  for r in range(128):
    for w in range(width // 16):
      zbuf[r, pl.ds(w * 16, 16)] = zero16

  def zloop(kk, carry):
    pltpu.sync_copy(zbuf, accum.at[pl.ds(si * _RPT + kk * 128, 128), :])
    return carry

  lax.fori_loop(0, _RPT // 128, zloop, 0)


def _sc_degree(dst2d):
  @functools.partial(
      pl.kernel,
      out_type=jax.ShapeDtypeStruct((2, _NPAD, 16), jnp.float32),
      mesh=_mesh,
      compiler_params=_sc_params,
      scratch_types=[
          pltpu.VMEM((_NCHUNK, 128), jnp.int32),
          pltpu.VMEM((128, 16), jnp.float32),
          pltpu.VMEM((128, 16), jnp.float32),
          pltpu.VMEM_SHARED((_NPAD, 16), jnp.float32),
          [pltpu.SemaphoreType.DMA] * 4,
          pltpu.SemaphoreType.DMA,
      ],
  )
  def k(dst_hbm, out_hbm, didx, ones, zbuf, accum, sems, semI):
    ci = lax.axis_index("c")
    si = lax.axis_index("s")
    wid = ci * 16 + si
    idx_cp = pltpu.make_async_copy(
        dst_hbm.at[pl.ds(wid * _NCHUNK, _NCHUNK), :], didx, semI)
    idx_cp.start()
    one16 = jnp.ones((16,), jnp.float32)
    for r in range(128):
      ones[r, pl.ds(0, 16)] = one16
    _zero_accum(zbuf, accum, si, 16)
    idx_cp.wait()
    plsc.subcore_barrier()

    def body(g, carry):
      for s in range(4):
        pltpu.async_copy(ones, accum.at[didx.at[4 * g + s]], sems[s], add=True)
      for s in range(4):
        pltpu.make_async_copy(ones, accum.at[didx.at[4 * g + s]], sems[s]).wait()
      return carry

    lax.fori_loop(0, _NGRP, body, 0)
    plsc.subcore_barrier()
    pltpu.sync_copy(accum.at[pl.ds(si * _RPT, _RPT), :],
                    out_hbm.at[ci, pl.ds(si * _RPT, _RPT), :])

  return k(dst2d)


def _sc_spmm_fused(zs1, dinv16, gidx2d, sidx2d):
  @functools.partial(
      pl.kernel,
      out_type=(jax.ShapeDtypeStruct((2, _NPAD, 32), jnp.float32),
                jax.ShapeDtypeStruct((2, _NPAD, 16), jnp.float32)),
      mesh=_mesh,
      compiler_params=_sc_params,
      scratch_types=[
          pltpu.VMEM((_NCHUNK, 128), jnp.int32),
          pltpu.VMEM((_NCHUNK, 128), jnp.int32),
          [pltpu.VMEM((128, 32), jnp.float32)] * 4,
          [pltpu.VMEM((128, 16), jnp.float32)] * 4,
          pltpu.VMEM((128, 32), jnp.float32),
          pltpu.VMEM((128, 16), jnp.float32),
          pltpu.VMEM_SHARED((_NPAD, 32), jnp.float32),
          pltpu.VMEM_SHARED((_NPAD, 16), jnp.float32),
          [pltpu.SemaphoreType.DMA] * 4,
          [pltpu.SemaphoreType.DMA] * 4,
          [pltpu.SemaphoreType.DMA] * 4,
          [pltpu.SemaphoreType.DMA] * 4,
          [pltpu.SemaphoreType.DMA] * 2,
      ],
  )
  def k(zs_hbm, dv_hbm, g_hbm, s_hbm, outA_hbm, outB_hbm, gi, si_, rowsA,
        rowsB, zbufA, zbufB, accumA, accumB, semAG, semAS, semBG, semBS, semI):
    ci = lax.axis_index("c")
    si = lax.axis_index("s")
    wid = ci * 16 + si
    gi_cp = pltpu.make_async_copy(
        g_hbm.at[pl.ds(wid * _NCHUNK, _NCHUNK), :], gi, semI[0])
    si_cp = pltpu.make_async_copy(
        s_hbm.at[pl.ds(wid * _NCHUNK, _NCHUNK), :], si_, semI[1])
    gi_cp.start()
    si_cp.start()
    _zero_accum(zbufA, accumA, si, 32)
    _zero_accum(zbufB, accumB, si, 16)
    gi_cp.wait()
    si_cp.wait()

    for s in range(4):
      pltpu.make_async_copy(zs_hbm.at[gi.at[s]], rowsA[s], semAG[s]).start()
      pltpu.make_async_copy(dv_hbm.at[si_.at[s]], rowsB[s], semBG[s]).start()
    plsc.subcore_barrier()

    def body(g, carry):
      for s in range(4):
        j = 4 * g + s
        pltpu.make_async_copy(zs_hbm.at[gi.at[j]], rowsA[s], semAG[s]).wait()
        pltpu.async_copy(rowsA[s], accumA.at[si_.at[j]], semAS[s], add=True)
        pltpu.make_async_copy(dv_hbm.at[si_.at[j]], rowsB[s], semBG[s]).wait()
        pltpu.async_copy(rowsB[s], accumB.at[gi.at[j]], semBS[s], add=True)

      @pl.when(g < _NGRP - 1)
      def _refill():
        for s in range(4):
          j = 4 * g + s
          pltpu.make_async_copy(rowsA[s], accumA.at[si_.at[j]], semAS[s]).wait()
          pltpu.make_async_copy(zs_hbm.at[gi.at[j + 4]], rowsA[s],
                                semAG[s]).start()
          pltpu.make_async_copy(rowsB[s], accumB.at[gi.at[j]], semBS[s]).wait()
          pltpu.make_async_copy(dv_hbm.at[si_.at[j + 4]], rowsB[s],
                                semBG[s]).start()

      return carry

    lax.fori_loop(0, _NGRP, body, 0)
    for s in range(4):
      j = _NCHUNK - 4 + s
      pltpu.make_async_copy(rowsA[s], accumA.at[si_.at[j]], semAS[s]).wait()
      pltpu.make_async_copy(rowsB[s], accumB.at[gi.at[j]], semBS[s]).wait()
    plsc.subcore_barrier()
    pltpu.sync_copy(accumA.at[pl.ds(si * _RPT, _RPT), :],
                    outA_hbm.at[ci, pl.ds(si * _RPT, _RPT), :])
    pltpu.sync_copy(accumB.at[pl.ds(si * _RPT, _RPT), :],
                    outB_hbm.at[ci, pl.ds(si * _RPT, _RPT), :])

  return k(zs1, dinv16, gidx2d, sidx2d)


def _sc_spmm(table, gidx2d, sidx2d):
  @functools.partial(
      pl.kernel,
      out_type=jax.ShapeDtypeStruct((2, _NPAD, 32), jnp.float32),
      mesh=_mesh,
      compiler_params=_sc_params,
      scratch_types=[
          pltpu.VMEM((_NCHUNK, 128), jnp.int32),
          pltpu.VMEM((_NCHUNK, 128), jnp.int32),
          [pltpu.VMEM((128, 32), jnp.float32)] * 4,
          pltpu.VMEM((128, 32), jnp.float32),
          pltpu.VMEM_SHARED((_NPAD, 32), jnp.float32),
          [pltpu.SemaphoreType.DMA] * 4,
          [pltpu.SemaphoreType.DMA] * 4,
          [pltpu.SemaphoreType.DMA] * 2,
      ],
  )
  def k(table_hbm, g_hbm, s_hbm, out_hbm, gi, si_, rows, zbuf, accum, semG,
        semS, semI):
    ci = lax.axis_index("c")
    si = lax.axis_index("s")
    wid = ci * 16 + si
    gi_cp = pltpu.make_async_copy(
        g_hbm.at[pl.ds(wid * _NCHUNK, _NCHUNK), :], gi, semI[0])
    si_cp = pltpu.make_async_copy(
        s_hbm.at[pl.ds(wid * _NCHUNK, _NCHUNK), :], si_, semI[1])
    gi_cp.start()
    si_cp.start()
    _zero_accum(zbuf, accum, si, 32)
    gi_cp.wait()
    si_cp.wait()

    for s in range(4):
      pltpu.make_async_copy(table_hbm.at[gi.at[s]], rows[s], semG[s]).start()
    plsc.subcore_barrier()

    def body(g, carry):
      for s in range(4):
        j = 4 * g + s
        pltpu.make_async_copy(table_hbm.at[gi.at[j]], rows[s], semG[s]).wait()
        pltpu.async_copy(rows[s], accum.at[si_.at[j]], semS[s], add=True)

      @pl.when(g < _NGRP - 1)
      def _refill():
        for s in range(4):
          j = 4 * g + s
          pltpu.make_async_copy(rows[s], accum.at[si_.at[j]], semS[s]).wait()
          pltpu.make_async_copy(table_hbm.at[gi.at[j + 4]], rows[s],
                                semG[s]).start()

      return carry

    lax.fori_loop(0, _NGRP, body, 0)
    for s in range(4):
      j = _NCHUNK - 4 + s
      pltpu.make_async_copy(rows[s], accum.at[si_.at[j]], semS[s]).wait()
    plsc.subcore_barrier()
    pltpu.sync_copy(accum.at[pl.ds(si * _RPT, _RPT), :],
                    out_hbm.at[ci, pl.ds(si * _RPT, _RPT), :])

  return k(table, gidx2d, sidx2d)


def _tc_mm1(xp, W1):
  R = 1024

  def body(x_ref, w_ref, o_ref):
    o_ref[...] = jnp.dot(x_ref[...], w_ref[...],
                         preferred_element_type=jnp.float32)

  return pl.pallas_call(
      body,
      grid=(_NPAD // R,),
      in_specs=[
          pl.BlockSpec((R, 128), lambda i: (i, 0)),
          pl.BlockSpec((128, 32), lambda i: (0, 0)),
      ],
      out_specs=pl.BlockSpec((R, 32), lambda i: (i, 0)),
      out_shape=jax.ShapeDtypeStruct((_NPAD, 32), jnp.float32),
  )(xp, W1)


def _tc_stage1(degp, xw1):
  R = 1024

  def body(dp_ref, xw_ref, zs_ref, dinv_ref, dinv16_ref):
    deg = dp_ref[0, :, 0:1] + dp_ref[1, :, 0:1] + 1.0
    dv = lax.rsqrt(deg)
    dinv_ref[...] = jnp.broadcast_to(dv, (R, 32))
    dinv16_ref[...] = jnp.broadcast_to(dv, (R, 16))
    zs_ref[...] = xw_ref[...] * dv

  return pl.pallas_call(
      body,
      grid=(_NPAD // R,),
      in_specs=[
          pl.BlockSpec((2, R, 16), lambda i: (0, i, 0)),
          pl.BlockSpec((R, 32), lambda i: (i, 0)),
      ],
      out_specs=[
          pl.BlockSpec((R, 32), lambda i: (i, 0)),
          pl.BlockSpec((R, 32), lambda i: (i, 0)),
          pl.BlockSpec((R, 16), lambda i: (i, 0)),
      ],
      out_shape=[
          jax.ShapeDtypeStruct((_NPAD, 32), jnp.float32),
          jax.ShapeDtypeStruct((_NPAD, 32), jnp.float32),
          jax.ShapeDtypeStruct((_NPAD, 16), jnp.float32),
      ],
  )(degp, xw1)


def _tc_stage2(part, zs1, dinv, b1, W2):
  R = 1024

  def body(p_ref, zs_ref, dv_ref, b_ref, w_ref, o_ref):
    s = p_ref[0] + p_ref[1] + zs_ref[...]
    h = jnp.maximum(dv_ref[...] * s + b_ref[...], 0.0)
    o_ref[...] = jnp.dot(h, w_ref[...],
                         preferred_element_type=jnp.float32) * dv_ref[...]

  return pl.pallas_call(
      body,
      grid=(_NPAD // R,),
      in_specs=[
          pl.BlockSpec((2, R, 32), lambda i: (0, i, 0)),
          pl.BlockSpec((R, 32), lambda i: (i, 0)),
          pl.BlockSpec((R, 32), lambda i: (i, 0)),
          pl.BlockSpec((1, 32), lambda i: (0, 0)),
          pl.BlockSpec((32, 32), lambda i: (0, 0)),
      ],
      out_specs=pl.BlockSpec((R, 32), lambda i: (i, 0)),
      out_shape=jax.ShapeDtypeStruct((_NPAD, 32), jnp.float32),
  )(part, zs1, dinv, b1, W2)


def _tc_stage3(part2, zs2, dinv, tp, b2, W3, b3, Wl, bl):
  def body(p_ref, zs_ref, dv_ref, t_ref, b2_ref, w3_ref, b3_ref, wl_ref,
           bl_ref, o_ref):
    s = p_ref[0] + p_ref[1] + zs_ref[...]
    h2 = jnp.maximum(dv_ref[...] * s + b2_ref[...], 0.0)
    dv1 = dv_ref[:, 0:1]
    t = t_ref[0, :, 0:1] + t_ref[1, :, 0:1]
    c = dv1 * t + dv1 * dv1
    row = lax.broadcasted_iota(jnp.int32, (_NPAD, 1), 0)
    c = jnp.where(row < _N, c, 0.0)
    r = jnp.sum(c * h2, axis=0, keepdims=True) * (1.0 / _N)
    g = jnp.dot(r, w3_ref[...], preferred_element_type=jnp.float32) + b3_ref[...]
    logits = jnp.dot(g, wl_ref[...], preferred_element_type=jnp.float32) + bl_ref[...]
    m = jnp.max(logits, axis=1, keepdims=True)
    e = jnp.exp(logits - m)
    o_ref[...] = e / jnp.sum(e, axis=1, keepdims=True)

  return pl.pallas_call(
      body,
      grid=(1,),
      in_specs=[
          pl.BlockSpec((2, _NPAD, 32), lambda i: (0, 0, 0)),
          pl.BlockSpec((_NPAD, 32), lambda i: (0, 0)),
          pl.BlockSpec((_NPAD, 32), lambda i: (0, 0)),
          pl.BlockSpec((2, _NPAD, 16), lambda i: (0, 0, 0)),
          pl.BlockSpec((1, 32), lambda i: (0, 0)),
          pl.BlockSpec((32, 32), lambda i: (0, 0)),
          pl.BlockSpec((1, 32), lambda i: (0, 0)),
          pl.BlockSpec((32, 3), lambda i: (0, 0)),
          pl.BlockSpec((1, 3), lambda i: (0, 0)),
      ],
      out_specs=pl.BlockSpec((1, 3), lambda i: (0, 0)),
      out_shape=jax.ShapeDtypeStruct((1, 3), jnp.float32),
  )(part2, zs2, dinv, tp, b2, W3, b3, Wl, bl)


def kernel(x, edge_index, W1, b1, W2, b2, W3, b3, Wl, bl):
  ei = edge_index.astype(jnp.int32)
  pad = _N + (jnp.arange(_EPAD - _E, dtype=jnp.int32) % (_NPAD - _N))
  src = jnp.concatenate([ei[0], pad]).reshape(_EPAD // 128, 128)
  dst = jnp.concatenate([ei[1], pad]).reshape(_EPAD // 128, 128)
  xp = jnp.pad(x, ((0, _NPAD - _N), (0, 0)))

  degp = _sc_degree(dst)
  xw1 = _tc_mm1(xp, W1)
  zs1, dinv, dinv16 = _tc_stage1(degp, xw1)
  p1, tp = _sc_spmm_fused(zs1, dinv16, src, dst)
  zs2 = _tc_stage2(p1, zs1, dinv, b1.reshape(1, 32), W2)
  p2 = _sc_spmm(zs2, src, dst)
  return _tc_stage3(p2, zs2, dinv, tp, b2.reshape(1, 32), W3,
                    b3.reshape(1, 32), Wl, bl.reshape(1, 3))


# speedup vs baseline: 1.0138x; 1.0138x over previous
import functools

import jax
import jax.numpy as jnp
from jax import lax
from jax.experimental import pallas as pl
from jax.experimental.pallas import tpu as pltpu
from jax.experimental.pallas import tpu_sc as plsc

_N = 10000
_NPAD = 10240
_E = 320000
_EPAD = 327680
_EPT = _EPAD // 32
_NCHUNK = _EPT // 128
_DEPTH = 4
_NGRP = _NCHUNK // _DEPTH
_RPT = _NPAD // 16

_mesh = plsc.VectorSubcoreMesh(core_axis_name="c", subcore_axis_name="s")
_sc_params = pltpu.CompilerParams(use_tc_tiling_on_sc=False)


def _zero_accum(zbuf, accum, si, width):
  zero16 = jnp.zeros((16,), jnp.float32)
  for r in range(128):
    for w in range(width // 16):
      zbuf[r, pl.ds(w * 16, 16)] = zero16

  def zloop(kk, carry):
    pltpu.sync_copy(zbuf, accum.at[pl.ds(si * _RPT + kk * 128, 128), :])
    return carry

  lax.fori_loop(0, _RPT // 128, zloop, 0)


def _sc_degree(dst2d):
  @functools.partial(
      pl.kernel,
      out_type=jax.ShapeDtypeStruct((2, _NPAD, 16), jnp.float32),
      mesh=_mesh,
      compiler_params=_sc_params,
      scratch_types=[
          pltpu.VMEM((_NCHUNK, 128), jnp.int32),
          pltpu.VMEM((128, 16), jnp.float32),
          pltpu.VMEM((128, 16), jnp.float32),
          pltpu.VMEM_SHARED((_NPAD, 16), jnp.float32),
          [pltpu.SemaphoreType.DMA] * _DEPTH,
          pltpu.SemaphoreType.DMA,
      ],
  )
  def k(dst_hbm, out_hbm, didx, ones, zbuf, accum, sems, semI):
    ci = lax.axis_index("c")
    si = lax.axis_index("s")
    wid = ci * 16 + si
    idx_cp = pltpu.make_async_copy(
        dst_hbm.at[pl.ds(wid * _NCHUNK, _NCHUNK), :], didx, semI)
    idx_cp.start()
    one16 = jnp.ones((16,), jnp.float32)
    for r in range(128):
      ones[r, pl.ds(0, 16)] = one16
    _zero_accum(zbuf, accum, si, 16)
    idx_cp.wait()
    plsc.subcore_barrier()

    def body(g, carry):
      for s in range(_DEPTH):
        pltpu.async_copy(ones, accum.at[didx.at[_DEPTH * g + s]], sems[s], add=True)
      for s in range(_DEPTH):
        pltpu.make_async_copy(ones, accum.at[didx.at[_DEPTH * g + s]], sems[s]).wait()
      return carry

    lax.fori_loop(0, _NGRP, body, 0)
    plsc.subcore_barrier()
    pltpu.sync_copy(accum.at[pl.ds(si * _RPT, _RPT), :],
                    out_hbm.at[ci, pl.ds(si * _RPT, _RPT), :])

  return k(dst2d)


def _sc_spmm_fused(zs1, dinv16, gidx2d, sidx2d):
  @functools.partial(
      pl.kernel,
      out_type=(jax.ShapeDtypeStruct((2, _NPAD, 32), jnp.float32),
                jax.ShapeDtypeStruct((2, _NPAD, 16), jnp.float32)),
      mesh=_mesh,
      compiler_params=_sc_params,
      scratch_types=[
          pltpu.VMEM((_NCHUNK, 128), jnp.int32),
          pltpu.VMEM((_NCHUNK, 128), jnp.int32),
          [pltpu.VMEM((128, 32), jnp.float32)] * _DEPTH,
          [pltpu.VMEM((128, 16), jnp.float32)] * _DEPTH,
          pltpu.VMEM((128, 32), jnp.float32),
          pltpu.VMEM((128, 16), jnp.float32),
          pltpu.VMEM_SHARED((_NPAD, 32), jnp.float32),
          pltpu.VMEM_SHARED((_NPAD, 16), jnp.float32),
          [pltpu.SemaphoreType.DMA] * _DEPTH,
          [pltpu.SemaphoreType.DMA] * _DEPTH,
          [pltpu.SemaphoreType.DMA] * _DEPTH,
          [pltpu.SemaphoreType.DMA] * _DEPTH,
          [pltpu.SemaphoreType.DMA] * 2,
      ],
  )
  def k(zs_hbm, dv_hbm, g_hbm, s_hbm, outA_hbm, outB_hbm, gi, si_, rowsA,
        rowsB, zbufA, zbufB, accumA, accumB, semAG, semAS, semBG, semBS, semI):
    ci = lax.axis_index("c")
    si = lax.axis_index("s")
    wid = ci * 16 + si
    gi_cp = pltpu.make_async_copy(
        g_hbm.at[pl.ds(wid * _NCHUNK, _NCHUNK), :], gi, semI[0])
    si_cp = pltpu.make_async_copy(
        s_hbm.at[pl.ds(wid * _NCHUNK, _NCHUNK), :], si_, semI[1])
    gi_cp.start()
    si_cp.start()
    _zero_accum(zbufA, accumA, si, 32)
    _zero_accum(zbufB, accumB, si, 16)
    gi_cp.wait()
    si_cp.wait()

    for s in range(_DEPTH):
      pltpu.make_async_copy(zs_hbm.at[gi.at[s]], rowsA[s], semAG[s]).start()
      pltpu.make_async_copy(dv_hbm.at[si_.at[s]], rowsB[s], semBG[s]).start()
    plsc.subcore_barrier()

    def body(g, carry):
      for s in range(_DEPTH):
        j = _DEPTH * g + s
        pltpu.make_async_copy(zs_hbm.at[gi.at[j]], rowsA[s], semAG[s]).wait()
        pltpu.async_copy(rowsA[s], accumA.at[si_.at[j]], semAS[s], add=True)
        pltpu.make_async_copy(dv_hbm.at[si_.at[j]], rowsB[s], semBG[s]).wait()
        pltpu.async_copy(rowsB[s], accumB.at[gi.at[j]], semBS[s], add=True)

      @pl.when(g < _NGRP - 1)
      def _refill():
        for s in range(_DEPTH):
          j = _DEPTH * g + s
          pltpu.make_async_copy(rowsA[s], accumA.at[si_.at[j]], semAS[s]).wait()
          pltpu.make_async_copy(zs_hbm.at[gi.at[j + _DEPTH]], rowsA[s],
                                semAG[s]).start()
          pltpu.make_async_copy(rowsB[s], accumB.at[gi.at[j]], semBS[s]).wait()
          pltpu.make_async_copy(dv_hbm.at[si_.at[j + _DEPTH]], rowsB[s],
                                semBG[s]).start()

      return carry

    lax.fori_loop(0, _NGRP, body, 0)
    for s in range(_DEPTH):
      j = _NCHUNK - _DEPTH + s
      pltpu.make_async_copy(rowsA[s], accumA.at[si_.at[j]], semAS[s]).wait()
      pltpu.make_async_copy(rowsB[s], accumB.at[gi.at[j]], semBS[s]).wait()
    plsc.subcore_barrier()
    pltpu.sync_copy(accumA.at[pl.ds(si * _RPT, _RPT), :],
                    outA_hbm.at[ci, pl.ds(si * _RPT, _RPT), :])
    pltpu.sync_copy(accumB.at[pl.ds(si * _RPT, _RPT), :],
                    outB_hbm.at[ci, pl.ds(si * _RPT, _RPT), :])

  return k(zs1, dinv16, gidx2d, sidx2d)


def _sc_spmm(table, gidx2d, sidx2d):
  @functools.partial(
      pl.kernel,
      out_type=jax.ShapeDtypeStruct((2, _NPAD, 32), jnp.float32),
      mesh=_mesh,
      compiler_params=_sc_params,
      scratch_types=[
          pltpu.VMEM((_NCHUNK, 128), jnp.int32),
          pltpu.VMEM((_NCHUNK, 128), jnp.int32),
          [pltpu.VMEM((128, 32), jnp.float32)] * _DEPTH,
          pltpu.VMEM((128, 32), jnp.float32),
          pltpu.VMEM_SHARED((_NPAD, 32), jnp.float32),
          [pltpu.SemaphoreType.DMA] * _DEPTH,
          [pltpu.SemaphoreType.DMA] * _DEPTH,
          [pltpu.SemaphoreType.DMA] * 2,
      ],
  )
  def k(table_hbm, g_hbm, s_hbm, out_hbm, gi, si_, rows, zbuf, accum, semG,
        semS, semI):
    ci = lax.axis_index("c")
    si = lax.axis_index("s")
    wid = ci * 16 + si
    gi_cp = pltpu.make_async_copy(
        g_hbm.at[pl.ds(wid * _NCHUNK, _NCHUNK), :], gi, semI[0])
    si_cp = pltpu.make_async_copy(
        s_hbm.at[pl.ds(wid * _NCHUNK, _NCHUNK), :], si_, semI[1])
    gi_cp.start()
    si_cp.start()
    _zero_accum(zbuf, accum, si, 32)
    gi_cp.wait()
    si_cp.wait()

    for s in range(_DEPTH):
      pltpu.make_async_copy(table_hbm.at[gi.at[s]], rows[s], semG[s]).start()
    plsc.subcore_barrier()

    def body(g, carry):
      for s in range(_DEPTH):
        j = _DEPTH * g + s
        pltpu.make_async_copy(table_hbm.at[gi.at[j]], rows[s], semG[s]).wait()
        pltpu.async_copy(rows[s], accum.at[si_.at[j]], semS[s], add=True)

      @pl.when(g < _NGRP - 1)
      def _refill():
        for s in range(_DEPTH):
          j = _DEPTH * g + s
          pltpu.make_async_copy(rows[s], accum.at[si_.at[j]], semS[s]).wait()
          pltpu.make_async_copy(table_hbm.at[gi.at[j + _DEPTH]], rows[s],
                                semG[s]).start()

      return carry

    lax.fori_loop(0, _NGRP, body, 0)
    for s in range(_DEPTH):
      j = _NCHUNK - _DEPTH + s
      pltpu.make_async_copy(rows[s], accum.at[si_.at[j]], semS[s]).wait()
    plsc.subcore_barrier()
    pltpu.sync_copy(accum.at[pl.ds(si * _RPT, _RPT), :],
                    out_hbm.at[ci, pl.ds(si * _RPT, _RPT), :])

  return k(table, gidx2d, sidx2d)


def _tc_stage1(degp, xp, W1):
  R = 1024

  def body(dp_ref, x_ref, w_ref, zs_ref, dinv_ref, dinv16_ref):
    deg = dp_ref[0, :, 0:1] + dp_ref[1, :, 0:1] + 1.0
    dv = lax.rsqrt(deg)
    dinv_ref[...] = jnp.broadcast_to(dv, (R, 32))
    dinv16_ref[...] = jnp.broadcast_to(dv, (R, 16))
    zs_ref[...] = jnp.dot(x_ref[...], w_ref[...],
                          preferred_element_type=jnp.float32) * dv

  return pl.pallas_call(
      body,
      grid=(_NPAD // R,),
      in_specs=[
          pl.BlockSpec((2, R, 16), lambda i: (0, i, 0)),
          pl.BlockSpec((R, 128), lambda i: (i, 0)),
          pl.BlockSpec((128, 32), lambda i: (0, 0)),
      ],
      out_specs=[
          pl.BlockSpec((R, 32), lambda i: (i, 0)),
          pl.BlockSpec((R, 32), lambda i: (i, 0)),
          pl.BlockSpec((R, 16), lambda i: (i, 0)),
      ],
      out_shape=[
          jax.ShapeDtypeStruct((_NPAD, 32), jnp.float32),
          jax.ShapeDtypeStruct((_NPAD, 32), jnp.float32),
          jax.ShapeDtypeStruct((_NPAD, 16), jnp.float32),
      ],
  )(degp, xp, W1)


def _tc_stage2(part, zs1, dinv, b1, W2):
  R = 1024

  def body(p_ref, zs_ref, dv_ref, b_ref, w_ref, o_ref):
    s = p_ref[0] + p_ref[1] + zs_ref[...]
    h = jnp.maximum(dv_ref[...] * s + b_ref[...], 0.0)
    o_ref[...] = jnp.dot(h, w_ref[...],
                         preferred_element_type=jnp.float32) * dv_ref[...]

  return pl.pallas_call(
      body,
      grid=(_NPAD // R,),
      in_specs=[
          pl.BlockSpec((2, R, 32), lambda i: (0, i, 0)),
          pl.BlockSpec((R, 32), lambda i: (i, 0)),
          pl.BlockSpec((R, 32), lambda i: (i, 0)),
          pl.BlockSpec((1, 32), lambda i: (0, 0)),
          pl.BlockSpec((32, 32), lambda i: (0, 0)),
      ],
      out_specs=pl.BlockSpec((R, 32), lambda i: (i, 0)),
      out_shape=jax.ShapeDtypeStruct((_NPAD, 32), jnp.float32),
  )(part, zs1, dinv, b1, W2)


def _tc_stage3(part2, zs2, dinv, tp, b2, W3, b3, Wl, bl):
  def body(p_ref, zs_ref, dv_ref, t_ref, b2_ref, w3_ref, b3_ref, wl_ref,
           bl_ref, o_ref):
    s = p_ref[0] + p_ref[1] + zs_ref[...]
    h2 = jnp.maximum(dv_ref[...] * s + b2_ref[...], 0.0)
    dv1 = dv_ref[:, 0:1]
    t = t_ref[0, :, 0:1] + t_ref[1, :, 0:1]
    c = dv1 * t + dv1 * dv1
    row = lax.broadcasted_iota(jnp.int32, (_NPAD, 1), 0)
    c = jnp.where(row < _N, c, 0.0)
    r = jnp.sum(c * h2, axis=0, keepdims=True) * (1.0 / _N)
    g = jnp.dot(r, w3_ref[...], preferred_element_type=jnp.float32) + b3_ref[...]
    logits = jnp.dot(g, wl_ref[...], preferred_element_type=jnp.float32) + bl_ref[...]
    m = jnp.max(logits, axis=1, keepdims=True)
    e = jnp.exp(logits - m)
    o_ref[...] = e / jnp.sum(e, axis=1, keepdims=True)

  return pl.pallas_call(
      body,
      grid=(1,),
      in_specs=[
          pl.BlockSpec((2, _NPAD, 32), lambda i: (0, 0, 0)),
          pl.BlockSpec((_NPAD, 32), lambda i: (0, 0)),
          pl.BlockSpec((_NPAD, 32), lambda i: (0, 0)),
          pl.BlockSpec((2, _NPAD, 16), lambda i: (0, 0, 0)),
          pl.BlockSpec((1, 32), lambda i: (0, 0)),
          pl.BlockSpec((32, 32), lambda i: (0, 0)),
          pl.BlockSpec((1, 32), lambda i: (0, 0)),
          pl.BlockSpec((32, 3), lambda i: (0, 0)),
          pl.BlockSpec((1, 3), lambda i: (0, 0)),
      ],
      out_specs=pl.BlockSpec((1, 3), lambda i: (0, 0)),
      out_shape=jax.ShapeDtypeStruct((1, 3), jnp.float32),
  )(part2, zs2, dinv, tp, b2, W3, b3, Wl, bl)


def kernel(x, edge_index, W1, b1, W2, b2, W3, b3, Wl, bl):
  ei = edge_index.astype(jnp.int32)
  pad = _N + (jnp.arange(_EPAD - _E, dtype=jnp.int32) % (_NPAD - _N))
  src = jnp.concatenate([ei[0], pad]).reshape(_EPAD // 128, 128)
  dst = jnp.concatenate([ei[1], pad]).reshape(_EPAD // 128, 128)
  xp = jnp.pad(x, ((0, _NPAD - _N), (0, 0)))

  degp = _sc_degree(dst)
  zs1, dinv, dinv16 = _tc_stage1(degp, xp, W1)
  p1, tp = _sc_spmm_fused(zs1, dinv16, src, dst)
  zs2 = _tc_stage2(p1, zs1, dinv, b1.reshape(1, 32), W2)
  p2 = _sc_spmm(zs2, src, dst)
  return _tc_stage3(p2, zs2, dinv, tp, b2.reshape(1, 32), W3,
                    b3.reshape(1, 32), Wl, bl.reshape(1, 3))
